# baseline (speedup 1.0000x reference)
import functools

import jax
import jax.numpy as jnp
from jax import lax
from jax.experimental import pallas as pl
from jax.experimental.pallas import tpu as pltpu
from jax.experimental.pallas import tpu_sc as plsc

N = 10000
E = 320000
IN_DIM = 128
HID = 32
LATENT = 16

NPAD = 10240
GRP = 128
NG = E // GRP
GPT = 78
CH = 6
NCHUNK = GPT // CH
RPT = NPAD // 16
DEGW = 8

_mesh = plsc.VectorSubcoreMesh(core_axis_name="c", subcore_axis_name="s")
_sc_params = pltpu.CompilerParams(use_tc_tiling_on_sc=False)


def _make_edge_scatter(F):
  @functools.partial(
      pl.kernel,
      out_type=jax.ShapeDtypeStruct((2, NPAD, F), jnp.float32),
      mesh=_mesh,
      compiler_params=_sc_params,
      scratch_types=[
          pltpu.VMEM((GPT, GRP), jnp.int32),
          pltpu.VMEM((GPT, GRP), jnp.int32),
          pltpu.VMEM((2, CH * GRP, F), jnp.float32),
          pltpu.VMEM((1, GRP), jnp.int32),
          pltpu.VMEM((1, GRP), jnp.int32),
          pltpu.VMEM_SHARED((NPAD, F), jnp.float32),
          pltpu.SemaphoreType.DMA,
          pltpu.SemaphoreType.DMA,
          pltpu.SemaphoreType.DMA,
      ],
  )
  def edge_scatter(g_hbm, srcg_hbm, dstg_hbm, zeros_hbm, out_hbm,
                   src_v, dst_v, rows_v, exs_v, exd_v, acc,
                   gsem0, gsem1, ssem):
    cid = lax.axis_index("c")
    sid = lax.axis_index("s")
    r0 = sid * RPT
    wid = cid * 16 + sid
    gbase = wid * GPT
    gsems = (gsem0, gsem1)

    pltpu.sync_copy(zeros_hbm.at[pl.ds(r0, RPT)], acc.at[pl.ds(r0, RPT)])
    pltpu.sync_copy(srcg_hbm.at[pl.ds(gbase, GPT)], src_v)
    pltpu.sync_copy(dstg_hbm.at[pl.ds(gbase, GPT)], dst_v)
    plsc.subcore_barrier()

    def issue_gathers(c, b):
      for j in range(CH):
        pltpu.async_copy(g_hbm.at[src_v.at[c * CH + j]],
                         rows_v.at[b].at[pl.ds(j * GRP, GRP)], gsems[b])

    def drain_gathers(b):
      pltpu.make_async_copy(g_hbm.at[pl.ds(0, CH * GRP)],
                            rows_v.at[b], gsems[b]).wait()

    def scatter_chunk(c, b):
      cps = [
          pltpu.async_copy(rows_v.at[b].at[pl.ds(j * GRP, GRP)],
                           acc.at[dst_v.at[c * CH + j]], ssem, add=True)
          for j in range(CH)
      ]
      for cp in cps:
        cp.wait()

    issue_gathers(0, 0)

    def pipe(kk, carry):
      c0 = 2 * kk
      issue_gathers(c0 + 1, 1)
      drain_gathers(0)
      scatter_chunk(c0, 0)
      issue_gathers(c0 + 2, 0)
      drain_gathers(1)
      scatter_chunk(c0 + 1, 1)
      return carry

    lax.fori_loop(0, (NCHUNK - 1) // 2, pipe, 0)
    drain_gathers(0)
    scatter_chunk(NCHUNK - 1, 0)

    @pl.when(wid < 4)
    def _():
      pltpu.sync_copy(srcg_hbm.at[pl.ds(2496 + wid, 1)], exs_v)
      pltpu.sync_copy(dstg_hbm.at[pl.ds(2496 + wid, 1)], exd_v)
      pltpu.async_copy(g_hbm.at[exs_v.at[0]],
                       rows_v.at[0].at[pl.ds(0, GRP)], gsem0).wait()
      pltpu.sync_copy(rows_v.at[0].at[pl.ds(0, GRP)],
                      acc.at[exd_v.at[0]], add=True)

    plsc.subcore_barrier()
    pltpu.sync_copy(acc.at[pl.ds(r0, RPT)], out_hbm.at[cid, pl.ds(r0, RPT)])

  return edge_scatter


_edge_scatter_hid = _make_edge_scatter(HID)
_edge_scatter_lat = _make_edge_scatter(LATENT)


@functools.partial(
    pl.kernel,
    out_type=jax.ShapeDtypeStruct((2, NPAD, DEGW), jnp.float32),
    mesh=_mesh,
    compiler_params=_sc_params,
    scratch_types=[
        pltpu.VMEM((GPT, GRP), jnp.int32),
        pltpu.VMEM((1, GRP), jnp.int32),
        pltpu.VMEM((GRP, DEGW), jnp.float32),
        pltpu.VMEM_SHARED((NPAD, DEGW), jnp.float32),
        pltpu.SemaphoreType.DMA,
    ],
)
def _deg_scatter(dstg_hbm, ones_hbm, zeros_hbm, out_hbm,
                 dst_v, exd_v, ones_v, acc, sem):
  cid = lax.axis_index("c")
  sid = lax.axis_index("s")
  r0 = sid * RPT
  wid = cid * 16 + sid
  pltpu.sync_copy(ones_hbm, ones_v)
  pltpu.sync_copy(zeros_hbm.at[pl.ds(r0, RPT)], acc.at[pl.ds(r0, RPT)])
  pltpu.sync_copy(dstg_hbm.at[pl.ds(wid * GPT, GPT)], dst_v)
  plsc.subcore_barrier()

  def grp_body(g, carry):
    pltpu.async_copy(ones_v, acc.at[dst_v.at[g]], sem, add=True)
    return carry

  lax.fori_loop(0, GPT, grp_body, 0)

  def drain_body(g, carry):
    pltpu.make_async_copy(ones_hbm, ones_v, sem).wait()
    return carry

  lax.fori_loop(0, GPT, drain_body, 0)

  @pl.when(wid < 4)
  def _():
    pltpu.sync_copy(dstg_hbm.at[pl.ds(2496 + wid, 1)], exd_v)
    pltpu.sync_copy(ones_v, acc.at[exd_v.at[0]], add=True)

  plsc.subcore_barrier()
  pltpu.sync_copy(acc.at[pl.ds(r0, RPT)], out_hbm.at[cid, pl.ds(r0, RPT)])


def _mm_body(x_ref, w_ref, h_ref):
  h_ref[...] = jnp.dot(x_ref[...], w_ref[...],
                       preferred_element_type=jnp.float32)


_mm1 = pl.pallas_call(
    _mm_body,
    grid=(10,),
    in_specs=[
        pl.BlockSpec((N // 10, IN_DIM), lambda i: (i, 0)),
        pl.BlockSpec((IN_DIM, HID), lambda i: (0, 0)),
    ],
    out_specs=pl.BlockSpec((N // 10, HID), lambda i: (i, 0)),
    out_shape=jax.ShapeDtypeStruct((N, HID), jnp.float32),
)

_mm2 = pl.pallas_call(
    _mm_body,
    grid=(10,),
    in_specs=[
        pl.BlockSpec((N // 10, HID), lambda i: (i, 0)),
        pl.BlockSpec((HID, LATENT), lambda i: (0, 0)),
    ],
    out_specs=pl.BlockSpec((N // 10, LATENT), lambda i: (i, 0)),
    out_shape=jax.ShapeDtypeStruct((N, LATENT), jnp.float32),
)


@jax.jit
def kernel(x, edge_index, W1, b1, W2, b2):
  srcg = edge_index[0].reshape(NG, GRP)
  dstg = edge_index[1].reshape(NG, GRP)

  ones = jnp.ones((GRP, DEGW), jnp.float32)
  zeros_d = jnp.zeros((NPAD, DEGW), jnp.float32)
  zeros_h = jnp.zeros((NPAD, HID), jnp.float32)
  zeros_l = jnp.zeros((NPAD, LATENT), jnp.float32)

  degp = _deg_scatter(dstg, ones, zeros_d)
  h = _mm1(x, W1)
  deg = degp[0, :, 0:1] + degp[1, :, 0:1] + 1.0
  dis = lax.rsqrt(deg)
  g1 = jnp.pad(h * dis[:N], ((0, NPAD - N), (0, 0)))

  p1 = _edge_scatter_hid(g1, srcg, dstg, zeros_h)
  s1 = p1[0, :N] + p1[1, :N] + g1[:N]
  out1 = jnp.maximum(dis[:N] * s1 + b1, 0.0)

  h2 = _mm2(out1, W2)
  g2 = jnp.pad(h2 * dis[:N], ((0, NPAD - N), (0, 0)))

  p2 = _edge_scatter_lat(g2, srcg, dstg, zeros_l)
  s2 = p2[0, :N] + p2[1, :N] + g2[:N]
  return dis[:N] * s2 + b2


# speedup vs baseline: 1.1636x; 1.1636x over previous
import functools

import jax
import jax.numpy as jnp
from jax import lax
from jax.experimental import pallas as pl
from jax.experimental.pallas import tpu as pltpu
from jax.experimental.pallas import tpu_sc as plsc

N = 10000
E = 320000
IN_DIM = 128
HID = 32
LATENT = 16

NPAD = 10240
GRP = 128
NG = E // GRP
GPT = 78
CH = 6
NCHUNK = GPT // CH
RPT = NPAD // 16
DEGW = 8

_mesh = plsc.VectorSubcoreMesh(core_axis_name="c", subcore_axis_name="s")
_sc_params = pltpu.CompilerParams(use_tc_tiling_on_sc=False)
_tc_params = pltpu.CompilerParams(skip_device_barrier=True)


def _make_edge_scatter(F):
  @functools.partial(
      pl.kernel,
      out_type=jax.ShapeDtypeStruct((2, NPAD, F), jnp.float32),
      mesh=_mesh,
      compiler_params=_sc_params,
      scratch_types=[
          pltpu.VMEM((GPT, GRP), jnp.int32),
          pltpu.VMEM((GPT, GRP), jnp.int32),
          pltpu.VMEM((2, CH * GRP, F), jnp.float32),
          pltpu.VMEM((1, GRP), jnp.int32),
          pltpu.VMEM((1, GRP), jnp.int32),
          pltpu.VMEM_SHARED((NPAD, F), jnp.float32),
          pltpu.SemaphoreType.DMA,
          pltpu.SemaphoreType.DMA,
          pltpu.SemaphoreType.DMA,
      ],
  )
  def edge_scatter(g_hbm, eidx_hbm, zeros_hbm, out_hbm,
                   src_v, dst_v, rows_v, exs_v, exd_v, acc,
                   gsem0, gsem1, ssem):
    cid = lax.axis_index("c")
    sid = lax.axis_index("s")
    r0 = sid * RPT
    wid = cid * 16 + sid
    gbase = wid * GPT
    gsems = (gsem0, gsem1)

    pltpu.sync_copy(zeros_hbm.at[pl.ds(r0, RPT)], acc.at[pl.ds(r0, RPT)])
    pltpu.sync_copy(eidx_hbm.at[pl.ds(gbase, GPT)], src_v)
    pltpu.sync_copy(eidx_hbm.at[pl.ds(NG + gbase, GPT)], dst_v)
    plsc.subcore_barrier()

    def issue_gathers(c, b):
      for j in range(CH):
        pltpu.async_copy(g_hbm.at[src_v.at[c * CH + j]],
                         rows_v.at[b].at[pl.ds(j * GRP, GRP)], gsems[b])

    def drain_gathers(b):
      pltpu.make_async_copy(g_hbm.at[pl.ds(0, CH * GRP)],
                            rows_v.at[b], gsems[b]).wait()

    def scatter_chunk(c, b):
      cps = [
          pltpu.async_copy(rows_v.at[b].at[pl.ds(j * GRP, GRP)],
                           acc.at[dst_v.at[c * CH + j]], ssem, add=True)
          for j in range(CH)
      ]
      for cp in cps:
        cp.wait()

    issue_gathers(0, 0)

    def pipe(kk, carry):
      c0 = 2 * kk
      issue_gathers(c0 + 1, 1)
      drain_gathers(0)
      scatter_chunk(c0, 0)
      issue_gathers(c0 + 2, 0)
      drain_gathers(1)
      scatter_chunk(c0 + 1, 1)
      return carry

    lax.fori_loop(0, (NCHUNK - 1) // 2, pipe, 0)
    drain_gathers(0)
    scatter_chunk(NCHUNK - 1, 0)

    @pl.when(wid < 4)
    def _():
      pltpu.sync_copy(eidx_hbm.at[pl.ds(2496 + wid, 1)], exs_v)
      pltpu.sync_copy(eidx_hbm.at[pl.ds(NG + 2496 + wid, 1)], exd_v)
      pltpu.async_copy(g_hbm.at[exs_v.at[0]],
                       rows_v.at[0].at[pl.ds(0, GRP)], gsem0).wait()
      pltpu.sync_copy(rows_v.at[0].at[pl.ds(0, GRP)],
                      acc.at[exd_v.at[0]], add=True)

    plsc.subcore_barrier()
    pltpu.sync_copy(acc.at[pl.ds(r0, RPT)], out_hbm.at[cid, pl.ds(r0, RPT)])

  return edge_scatter


_edge_scatter_hid = _make_edge_scatter(HID)
_edge_scatter_lat = _make_edge_scatter(LATENT)


@functools.partial(
    pl.kernel,
    out_type=jax.ShapeDtypeStruct((2, NPAD, DEGW), jnp.float32),
    mesh=_mesh,
    compiler_params=_sc_params,
    scratch_types=[
        pltpu.VMEM((GPT, GRP), jnp.int32),
        pltpu.VMEM((1, GRP), jnp.int32),
        pltpu.VMEM((GRP, DEGW), jnp.float32),
        pltpu.VMEM_SHARED((NPAD, DEGW), jnp.float32),
        pltpu.SemaphoreType.DMA,
    ],
)
def _deg_scatter(eidx_hbm, ones_hbm, zeros_hbm, out_hbm,
                 dst_v, exd_v, ones_v, acc, sem):
  cid = lax.axis_index("c")
  sid = lax.axis_index("s")
  r0 = sid * RPT
  wid = cid * 16 + sid
  pltpu.sync_copy(ones_hbm, ones_v)
  pltpu.sync_copy(zeros_hbm.at[pl.ds(r0, RPT)], acc.at[pl.ds(r0, RPT)])
  pltpu.sync_copy(eidx_hbm.at[pl.ds(NG + wid * GPT, GPT)], dst_v)
  plsc.subcore_barrier()

  def grp_body(g, carry):
    pltpu.async_copy(ones_v, acc.at[dst_v.at[g]], sem, add=True)
    return carry

  lax.fori_loop(0, GPT, grp_body, 0)

  def drain_body(g, carry):
    pltpu.make_async_copy(ones_hbm, ones_v, sem).wait()
    return carry

  lax.fori_loop(0, GPT, drain_body, 0)

  @pl.when(wid < 4)
  def _():
    pltpu.sync_copy(eidx_hbm.at[pl.ds(NG + 2496 + wid, 1)], exd_v)
    pltpu.sync_copy(ones_v, acc.at[exd_v.at[0]], add=True)

  plsc.subcore_barrier()
  pltpu.sync_copy(acc.at[pl.ds(r0, RPT)], out_hbm.at[cid, pl.ds(r0, RPT)])


def _mm1_body(x_ref, w1_ref, h_ref):
  h_ref[...] = jnp.dot(x_ref[...], w1_ref[...],
                       preferred_element_type=jnp.float32)


_mm1 = pl.pallas_call(
    _mm1_body,
    grid=(10,),
    in_specs=[
        pl.BlockSpec((N // 10, IN_DIM), lambda i: (i, 0)),
        pl.BlockSpec((IN_DIM, HID), lambda i: (0, 0)),
    ],
    out_specs=pl.BlockSpec((N // 10, HID), lambda i: (i, 0)),
    out_shape=jax.ShapeDtypeStruct((N, HID), jnp.float32),
    compiler_params=_tc_params,
)


def _tc1_body(h_ref, degp_ref, g1_ref, dis_ref):
  deg = degp_ref[0][:, 0:1] + degp_ref[1][:, 0:1] + 1.0
  dis = lax.rsqrt(deg)
  dis_ref[...] = dis
  g1_ref[0:N, :] = h_ref[...] * dis[0:N]


_tc1 = pl.pallas_call(
    _tc1_body,
    out_shape=[
        jax.ShapeDtypeStruct((NPAD, HID), jnp.float32),
        jax.ShapeDtypeStruct((NPAD, 1), jnp.float32),
    ],
    compiler_params=_tc_params,
)


def _tc2_body(p_ref, g1_ref, dis_ref, b1_ref, w2_ref, g2_ref):
  s = p_ref[0][0:N] + p_ref[1][0:N] + g1_ref[0:N]
  out1 = jnp.maximum(dis_ref[0:N] * s + b1_ref[...], 0.0)
  h2 = jnp.dot(out1, w2_ref[...], preferred_element_type=jnp.float32)
  g2_ref[0:N, :] = h2 * dis_ref[0:N]


_tc2 = pl.pallas_call(
    _tc2_body,
    out_shape=jax.ShapeDtypeStruct((NPAD, LATENT), jnp.float32),
    compiler_params=_tc_params,
)


def _tc3_body(p_ref, g2_ref, dis_ref, b2_ref, out_ref):
  s = p_ref[0][0:N] + p_ref[1][0:N] + g2_ref[0:N]
  out_ref[...] = dis_ref[0:N] * s + b2_ref[...]


_tc3 = pl.pallas_call(
    _tc3_body,
    out_shape=jax.ShapeDtypeStruct((N, LATENT), jnp.float32),
    compiler_params=_tc_params,
)


@jax.jit
def kernel(x, edge_index, W1, b1, W2, b2):
  eidx = edge_index.reshape(2 * NG, GRP)

  ones = jnp.ones((GRP, DEGW), jnp.float32)
  zeros_d = jnp.zeros((NPAD, DEGW), jnp.float32)
  zeros_h = jnp.zeros((NPAD, HID), jnp.float32)
  zeros_l = jnp.zeros((NPAD, LATENT), jnp.float32)

  degp = _deg_scatter(eidx, ones, zeros_d)
  h = _mm1(x, W1)
  g1, dis = _tc1(h, degp)
  p1 = _edge_scatter_hid(g1, eidx, zeros_h)
  g2 = _tc2(p1, g1, dis, b1.reshape(1, HID), W2)
  p2 = _edge_scatter_lat(g2, eidx, zeros_l)
  return _tc3(p2, g2, dis, b2.reshape(1, LATENT))


# speedup vs baseline: 1.2125x; 1.0420x over previous
import functools

import jax
import jax.numpy as jnp
from jax import lax
from jax.experimental import pallas as pl
from jax.experimental.pallas import tpu as pltpu
from jax.experimental.pallas import tpu_sc as plsc

N = 10000
E = 320000
IN_DIM = 128
HID = 32
LATENT = 16

NPAD = 10240
GRP = 128
NG = E // GRP
GPT = 78
CH = 6
NCHUNK = GPT // CH
RPT = NPAD // 16
DEGW = 8

_mesh = plsc.VectorSubcoreMesh(core_axis_name="c", subcore_axis_name="s")
_sc_params = pltpu.CompilerParams(use_tc_tiling_on_sc=False)
_tc_params = pltpu.CompilerParams(skip_device_barrier=True)


def _make_edge_scatter(F):
  @functools.partial(
      pl.kernel,
      out_type=jax.ShapeDtypeStruct((2, NPAD, F), jnp.float32),
      mesh=_mesh,
      compiler_params=_sc_params,
      scratch_types=[
          pltpu.VMEM((GPT, GRP), jnp.int32),
          pltpu.VMEM((GPT, GRP), jnp.int32),
          pltpu.VMEM((2, CH * GRP, F), jnp.float32),
          pltpu.VMEM((1, GRP), jnp.int32),
          pltpu.VMEM((1, GRP), jnp.int32),
          pltpu.VMEM_SHARED((NPAD, F), jnp.float32),
          pltpu.SemaphoreType.DMA,
          pltpu.SemaphoreType.DMA,
          pltpu.SemaphoreType.DMA,
      ],
  )
  def edge_scatter(g_hbm, eidx_hbm, zeros_hbm, out_hbm,
                   src_v, dst_v, rows_v, exs_v, exd_v, acc,
                   gsem0, gsem1, ssem):
    cid = lax.axis_index("c")
    sid = lax.axis_index("s")
    r0 = sid * RPT
    wid = cid * 16 + sid
    gbase = wid * GPT
    gsems = (gsem0, gsem1)

    pltpu.sync_copy(zeros_hbm.at[pl.ds(r0, RPT)], acc.at[pl.ds(r0, RPT)])
    pltpu.sync_copy(eidx_hbm.at[pl.ds(gbase, GPT)], src_v)
    pltpu.sync_copy(eidx_hbm.at[pl.ds(NG + gbase, GPT)], dst_v)
    plsc.subcore_barrier()

    def issue_gathers(c, b):
      for j in range(CH):
        pltpu.async_copy(g_hbm.at[src_v.at[c * CH + j]],
                         rows_v.at[b].at[pl.ds(j * GRP, GRP)], gsems[b])

    def drain_gathers(b):
      pltpu.make_async_copy(g_hbm.at[pl.ds(0, CH * GRP)],
                            rows_v.at[b], gsems[b]).wait()

    def scatter_chunk(c, b):
      cps = [
          pltpu.async_copy(rows_v.at[b].at[pl.ds(j * GRP, GRP)],
                           acc.at[dst_v.at[c * CH + j]], ssem, add=True)
          for j in range(CH)
      ]
      for cp in cps:
        cp.wait()

    issue_gathers(0, 0)

    def pipe(kk, carry):
      c0 = 2 * kk
      issue_gathers(c0 + 1, 1)
      drain_gathers(0)
      scatter_chunk(c0, 0)
      issue_gathers(c0 + 2, 0)
      drain_gathers(1)
      scatter_chunk(c0 + 1, 1)
      return carry

    lax.fori_loop(0, (NCHUNK - 1) // 2, pipe, 0)
    drain_gathers(0)
    scatter_chunk(NCHUNK - 1, 0)

    @pl.when(wid < 4)
    def _():
      pltpu.sync_copy(eidx_hbm.at[pl.ds(2496 + wid, 1)], exs_v)
      pltpu.sync_copy(eidx_hbm.at[pl.ds(NG + 2496 + wid, 1)], exd_v)
      pltpu.async_copy(g_hbm.at[exs_v.at[0]],
                       rows_v.at[0].at[pl.ds(0, GRP)], gsem0).wait()
      pltpu.sync_copy(rows_v.at[0].at[pl.ds(0, GRP)],
                      acc.at[exd_v.at[0]], add=True)

    plsc.subcore_barrier()
    pltpu.sync_copy(acc.at[pl.ds(r0, RPT)], out_hbm.at[cid, pl.ds(r0, RPT)])

  return edge_scatter


_edge_scatter_hid = _make_edge_scatter(HID)
_edge_scatter_lat = _make_edge_scatter(LATENT)


@functools.partial(
    pl.kernel,
    out_type=jax.ShapeDtypeStruct((2, NPAD, DEGW), jnp.float32),
    mesh=_mesh,
    compiler_params=_sc_params,
    scratch_types=[
        pltpu.VMEM((GPT, GRP), jnp.int32),
        pltpu.VMEM((1, GRP), jnp.int32),
        pltpu.VMEM((GRP, DEGW), jnp.float32),
        pltpu.VMEM_SHARED((NPAD, DEGW), jnp.float32),
        pltpu.SemaphoreType.DMA,
    ],
)
def _deg_scatter(eidx_hbm, ones_hbm, zeros_hbm, out_hbm,
                 dst_v, exd_v, ones_v, acc, sem):
  cid = lax.axis_index("c")
  sid = lax.axis_index("s")
  r0 = sid * RPT
  wid = cid * 16 + sid
  pltpu.sync_copy(ones_hbm, ones_v)
  pltpu.sync_copy(zeros_hbm.at[pl.ds(r0, RPT)], acc.at[pl.ds(r0, RPT)])
  pltpu.sync_copy(eidx_hbm.at[pl.ds(NG + wid * GPT, GPT)], dst_v)
  plsc.subcore_barrier()

  def grp_body(g, carry):
    pltpu.async_copy(ones_v, acc.at[dst_v.at[g]], sem, add=True)
    return carry

  lax.fori_loop(0, GPT, grp_body, 0)

  def drain_body(g, carry):
    pltpu.make_async_copy(ones_hbm, ones_v, sem).wait()
    return carry

  lax.fori_loop(0, GPT, drain_body, 0)

  @pl.when(wid < 4)
  def _():
    pltpu.sync_copy(eidx_hbm.at[pl.ds(NG + 2496 + wid, 1)], exd_v)
    pltpu.sync_copy(ones_v, acc.at[exd_v.at[0]], add=True)

  plsc.subcore_barrier()
  pltpu.sync_copy(acc.at[pl.ds(r0, RPT)], out_hbm.at[cid, pl.ds(r0, RPT)])


def _mm1_body(x_ref, w1_ref, h_ref):
  h_ref[...] = jnp.dot(x_ref[...], w1_ref[...],
                       preferred_element_type=jnp.float32)


_mm1 = pl.pallas_call(
    _mm1_body,
    grid=(10,),
    in_specs=[
        pl.BlockSpec((N // 10, IN_DIM), lambda i: (i, 0)),
        pl.BlockSpec((IN_DIM, HID), lambda i: (0, 0)),
    ],
    out_specs=pl.BlockSpec((N // 10, HID), lambda i: (i, 0)),
    out_shape=jax.ShapeDtypeStruct((N, HID), jnp.float32),
    compiler_params=_tc_params,
)


def _tc1_body(h_ref, degp_ref, g1_ref, dis_ref):
  deg = degp_ref[0][:, 0:1] + degp_ref[1][:, 0:1] + 1.0
  dis = lax.rsqrt(deg)
  dis_ref[...] = dis
  g1_ref[0:N, :] = h_ref[...] * dis[0:N]


_tc1 = pl.pallas_call(
    _tc1_body,
    out_shape=[
        jax.ShapeDtypeStruct((NPAD, HID), jnp.float32),
        jax.ShapeDtypeStruct((NPAD, 1), jnp.float32),
    ],
    compiler_params=pltpu.CompilerParams(
        skip_device_barrier=True, allow_input_fusion=(True, True)),
)


def _tc2_body(p_ref, g1_ref, dis_ref, b1_ref, w2_ref, g2_ref):
  s = p_ref[0][0:N] + p_ref[1][0:N] + g1_ref[0:N]
  out1 = jnp.maximum(dis_ref[0:N] * s + b1_ref[...], 0.0)
  h2 = jnp.dot(out1, w2_ref[...], preferred_element_type=jnp.float32)
  g2_ref[0:N, :] = h2 * dis_ref[0:N]


_tc2 = pl.pallas_call(
    _tc2_body,
    out_shape=jax.ShapeDtypeStruct((NPAD, LATENT), jnp.float32),
    compiler_params=pltpu.CompilerParams(
        skip_device_barrier=True,
        allow_input_fusion=(True, True, True, True, True)),
)


def _tc3_body(p_ref, g2_ref, dis_ref, b2_ref, out_ref):
  s = p_ref[0][0:N] + p_ref[1][0:N] + g2_ref[0:N]
  out_ref[...] = dis_ref[0:N] * s + b2_ref[...]


_tc3 = pl.pallas_call(
    _tc3_body,
    out_shape=jax.ShapeDtypeStruct((N, LATENT), jnp.float32),
    compiler_params=pltpu.CompilerParams(
        skip_device_barrier=True,
        allow_input_fusion=(True, True, True, True)),
)


@jax.jit
def kernel(x, edge_index, W1, b1, W2, b2):
  eidx = edge_index.reshape(2 * NG, GRP)

  ones = jnp.ones((GRP, DEGW), jnp.float32)
  zeros_d = jnp.zeros((NPAD, DEGW), jnp.float32)
  zeros_h = jnp.zeros((NPAD, HID), jnp.float32)
  zeros_l = jnp.zeros((NPAD, LATENT), jnp.float32)

  degp = _deg_scatter(eidx, ones, zeros_d)
  h = _mm1(x, W1)
  g1, dis = _tc1(h, degp)
  p1 = _edge_scatter_hid(g1, eidx, zeros_h)
  g2 = _tc2(p1, g1, dis, b1.reshape(1, HID), W2)
  p2 = _edge_scatter_lat(g2, eidx, zeros_l)
  return _tc3(p2, g2, dis, b2.reshape(1, LATENT))


# speedup vs baseline: 1.2216x; 1.0075x over previous
import functools

import jax
import jax.numpy as jnp
from jax import lax
from jax.experimental import pallas as pl
from jax.experimental.pallas import tpu as pltpu
from jax.experimental.pallas import tpu_sc as plsc

N = 10000
E = 320000
IN_DIM = 128
HID = 32
LATENT = 16

NPAD = 10240
GRP = 128
NG = E // GRP
GPT = 78
RPT = NPAD // 16
DEGW = 8

_mesh = plsc.VectorSubcoreMesh(core_axis_name="c", subcore_axis_name="s")
_sc_params = pltpu.CompilerParams(use_tc_tiling_on_sc=False)
_tc_params = pltpu.CompilerParams(skip_device_barrier=True)


def _make_edge_scatter(F, CH):
  NCHUNK = GPT // CH

  @functools.partial(
      pl.kernel,
      out_type=jax.ShapeDtypeStruct((2, NPAD, F), jnp.float32),
      mesh=_mesh,
      compiler_params=_sc_params,
      scratch_types=[
          pltpu.VMEM((GPT, GRP), jnp.int32),
          pltpu.VMEM((GPT, GRP), jnp.int32),
          pltpu.VMEM((2, CH * GRP, F), jnp.float32),
          pltpu.VMEM((1, GRP), jnp.int32),
          pltpu.VMEM((1, GRP), jnp.int32),
          pltpu.VMEM_SHARED((NPAD, F), jnp.float32),
          pltpu.SemaphoreType.DMA,
          pltpu.SemaphoreType.DMA,
          pltpu.SemaphoreType.DMA,
      ],
  )
  def edge_scatter(g_hbm, eidx_hbm, zeros_hbm, out_hbm,
                   src_v, dst_v, rows_v, exs_v, exd_v, acc,
                   gsem0, gsem1, ssem):
    cid = lax.axis_index("c")
    sid = lax.axis_index("s")
    r0 = sid * RPT
    wid = cid * 16 + sid
    gbase = wid * GPT
    gsems = (gsem0, gsem1)

    pltpu.sync_copy(zeros_hbm.at[pl.ds(r0, RPT)], acc.at[pl.ds(r0, RPT)])
    pltpu.sync_copy(eidx_hbm.at[pl.ds(gbase, GPT)], src_v)
    pltpu.sync_copy(eidx_hbm.at[pl.ds(NG + gbase, GPT)], dst_v)
    plsc.subcore_barrier()

    def issue_gathers(c, b):
      for j in range(CH):
        pltpu.async_copy(g_hbm.at[src_v.at[c * CH + j]],
                         rows_v.at[b].at[pl.ds(j * GRP, GRP)], gsems[b])

    def drain_gathers(b):
      pltpu.make_async_copy(g_hbm.at[pl.ds(0, CH * GRP)],
                            rows_v.at[b], gsems[b]).wait()

    def scatter_chunk(c, b):
      cps = [
          pltpu.async_copy(rows_v.at[b].at[pl.ds(j * GRP, GRP)],
                           acc.at[dst_v.at[c * CH + j]], ssem, add=True)
          for j in range(CH)
      ]
      for cp in cps:
        cp.wait()

    issue_gathers(0, 0)

    def pipe(kk, carry):
      c0 = 2 * kk
      issue_gathers(c0 + 1, 1)
      drain_gathers(0)
      scatter_chunk(c0, 0)
      issue_gathers(c0 + 2, 0)
      drain_gathers(1)
      scatter_chunk(c0 + 1, 1)
      return carry

    if NCHUNK % 2:
      lax.fori_loop(0, (NCHUNK - 1) // 2, pipe, 0)
      drain_gathers(0)
      scatter_chunk(NCHUNK - 1, 0)
    else:
      lax.fori_loop(0, (NCHUNK - 2) // 2, pipe, 0)
      issue_gathers(NCHUNK - 1, 1)
      drain_gathers(0)
      scatter_chunk(NCHUNK - 2, 0)
      drain_gathers(1)
      scatter_chunk(NCHUNK - 1, 1)

    @pl.when(wid < 4)
    def _():
      pltpu.sync_copy(eidx_hbm.at[pl.ds(2496 + wid, 1)], exs_v)
      pltpu.sync_copy(eidx_hbm.at[pl.ds(NG + 2496 + wid, 1)], exd_v)
      pltpu.async_copy(g_hbm.at[exs_v.at[0]],
                       rows_v.at[0].at[pl.ds(0, GRP)], gsem0).wait()
      pltpu.sync_copy(rows_v.at[0].at[pl.ds(0, GRP)],
                      acc.at[exd_v.at[0]], add=True)

    plsc.subcore_barrier()
    pltpu.sync_copy(acc.at[pl.ds(r0, RPT)], out_hbm.at[cid, pl.ds(r0, RPT)])

  return edge_scatter


_edge_scatter_hid = _make_edge_scatter(HID, 6)
_edge_scatter_lat = _make_edge_scatter(LATENT, 13)


@functools.partial(
    pl.kernel,
    out_type=jax.ShapeDtypeStruct((2, NPAD, DEGW), jnp.float32),
    mesh=_mesh,
    compiler_params=_sc_params,
    scratch_types=[
        pltpu.VMEM((GPT, GRP), jnp.int32),
        pltpu.VMEM((1, GRP), jnp.int32),
        pltpu.VMEM((GRP, DEGW), jnp.float32),
        pltpu.VMEM_SHARED((NPAD, DEGW), jnp.float32),
        pltpu.SemaphoreType.DMA,
    ],
)
def _deg_scatter(eidx_hbm, ones_hbm, zeros_hbm, out_hbm,
                 dst_v, exd_v, ones_v, acc, sem):
  cid = lax.axis_index("c")
  sid = lax.axis_index("s")
  r0 = sid * RPT
  wid = cid * 16 + sid
  pltpu.sync_copy(ones_hbm, ones_v)
  pltpu.sync_copy(zeros_hbm.at[pl.ds(r0, RPT)], acc.at[pl.ds(r0, RPT)])
  pltpu.sync_copy(eidx_hbm.at[pl.ds(NG + wid * GPT, GPT)], dst_v)
  plsc.subcore_barrier()

  def grp_body(g, carry):
    pltpu.async_copy(ones_v, acc.at[dst_v.at[g]], sem, add=True)
    return carry

  lax.fori_loop(0, GPT, grp_body, 0)

  def drain_body(g, carry):
    pltpu.make_async_copy(ones_hbm, ones_v, sem).wait()
    return carry

  lax.fori_loop(0, GPT, drain_body, 0)

  @pl.when(wid < 4)
  def _():
    pltpu.sync_copy(eidx_hbm.at[pl.ds(NG + 2496 + wid, 1)], exd_v)
    pltpu.sync_copy(ones_v, acc.at[exd_v.at[0]], add=True)

  plsc.subcore_barrier()
  pltpu.sync_copy(acc.at[pl.ds(r0, RPT)], out_hbm.at[cid, pl.ds(r0, RPT)])


def _mm1_body(x_ref, w1_ref, h_ref):
  h_ref[...] = jnp.dot(x_ref[...], w1_ref[...],
                       preferred_element_type=jnp.float32)


_mm1 = pl.pallas_call(
    _mm1_body,
    grid=(10,),
    in_specs=[
        pl.BlockSpec((N // 10, IN_DIM), lambda i: (i, 0)),
        pl.BlockSpec((IN_DIM, HID), lambda i: (0, 0)),
    ],
    out_specs=pl.BlockSpec((N // 10, HID), lambda i: (i, 0)),
    out_shape=jax.ShapeDtypeStruct((N, HID), jnp.float32),
    compiler_params=_tc_params,
)


def _tc1_body(h_ref, degp_ref, g1_ref, dis_ref):
  deg = degp_ref[0][:, 0:1] + degp_ref[1][:, 0:1] + 1.0
  dis = lax.rsqrt(deg)
  dis_ref[...] = dis
  g1_ref[0:N, :] = h_ref[...] * dis[0:N]


_tc1 = pl.pallas_call(
    _tc1_body,
    out_shape=[
        jax.ShapeDtypeStruct((NPAD, HID), jnp.float32),
        jax.ShapeDtypeStruct((NPAD, 1), jnp.float32),
    ],
    compiler_params=pltpu.CompilerParams(
        skip_device_barrier=True, allow_input_fusion=(True, True)),
)


def _tc2_body(p_ref, g1_ref, dis_ref, b1_ref, w2_ref, g2_ref):
  s = p_ref[0][0:N] + p_ref[1][0:N] + g1_ref[0:N]
  out1 = jnp.maximum(dis_ref[0:N] * s + b1_ref[...], 0.0)
  h2 = jnp.dot(out1, w2_ref[...], preferred_element_type=jnp.float32)
  g2_ref[0:N, :] = h2 * dis_ref[0:N]


_tc2 = pl.pallas_call(
    _tc2_body,
    out_shape=jax.ShapeDtypeStruct((NPAD, LATENT), jnp.float32),
    compiler_params=pltpu.CompilerParams(
        skip_device_barrier=True,
        allow_input_fusion=(True, True, True, True, True)),
)


def _tc3_body(p_ref, g2_ref, dis_ref, b2_ref, out_ref):
  s = p_ref[0][0:N] + p_ref[1][0:N] + g2_ref[0:N]
  out_ref[...] = dis_ref[0:N] * s + b2_ref[...]


_tc3 = pl.pallas_call(
    _tc3_body,
    out_shape=jax.ShapeDtypeStruct((N, LATENT), jnp.float32),
    compiler_params=pltpu.CompilerParams(
        skip_device_barrier=True,
        allow_input_fusion=(True, True, True, True)),
)


@jax.jit
def kernel(x, edge_index, W1, b1, W2, b2):
  eidx = edge_index.reshape(2 * NG, GRP)

  ones = jnp.ones((GRP, DEGW), jnp.float32)
  zeros_d = jnp.zeros((NPAD, DEGW), jnp.float32)
  zeros_h = jnp.zeros((NPAD, HID), jnp.float32)
  zeros_l = jnp.zeros((NPAD, LATENT), jnp.float32)

  degp = _deg_scatter(eidx, ones, zeros_d)
  h = _mm1(x, W1)
  g1, dis = _tc1(h, degp)
  p1 = _edge_scatter_hid(g1, eidx, zeros_h)
  g2 = _tc2(p1, g1, dis, b1.reshape(1, HID), W2)
  p2 = _edge_scatter_lat(g2, eidx, zeros_l)
  return _tc3(p2, g2, dis, b2.reshape(1, LATENT))
